# baseline (device time: 205425 ns/iter reference)
import functools

import jax
import jax.numpy as jnp
from jax import lax
from jax.experimental import pallas as pl
from jax.experimental.pallas import tpu as pltpu

N_DEV = 16
Z = 4
Q = 4
C = 8


def kernel(x, w_mat):
    m_glob, k_per = x.shape
    _, n = w_mat.shape
    m_per = m_glob // N_DEV
    slice_w = n // C
    half_w = slice_w // 2

    def body(x_ref, w_ref, out_ref, *scratch):
        bufR, bufL = scratch[0], scratch[1]
        ssemR, rsemR = scratch[2], scratch[3]
        ssemL, rsemL = scratch[4], scratch[5]
        p2buf = scratch[6]
        p2ssem, p2rsem = scratch[7], scratch[8]

        my = lax.axis_index("i")
        myz = my // Q
        myq = lax.rem(my, Q)
        plane_base = myz * Q
        q_right = plane_base + lax.rem(myq + 1, Q)
        q_left = plane_base + lax.rem(myq + Q - 1, Q)

        partners = [q_left, q_right]
        for k in range(Z - 1):
            dz = jnp.where(myz <= k, k + 1, k)
            partners.append(dz * Q + myq)
        barrier_sem = pltpu.get_barrier_semaphore()
        for nbr in partners:
            pl.semaphore_signal(
                barrier_sem, inc=1,
                device_id=(nbr,), device_id_type=pl.DeviceIdType.MESH,
            )
        pl.semaphore_wait(barrier_sem, len(partners))

        sends = []

        def phase1_step(j, s):
            col0 = j * slice_w
            for dirn in (+1, -1):
                if dirn == +1:
                    g = lax.rem(myq + 2 * Q - 1 - s, Q)
                    src_nbr, dst_nbr = q_left, q_right
                    buf, ssem, rsem = bufR, ssemR, rsemR
                    c0 = col0
                else:
                    g = lax.rem(myq + 1 + s, Q)
                    src_nbr, dst_nbr = q_right, q_left
                    buf, ssem, rsem = bufL, ssemL, rsemL
                    c0 = col0 + half_w
                parts = []
                for zb in range(Z):
                    b = zb * Q + g
                    parts.append(
                        jnp.dot(
                            x_ref[pl.ds(b * m_per, m_per), :],
                            w_ref[:, c0 : c0 + half_w],
                            preferred_element_type=jnp.float32,
                        )
                    )
                slot = 3 if s == 0 else s - 1
                if s == 0:
                    for zb in range(Z):
                        buf[j, slot, zb * m_per : (zb + 1) * m_per, :] = (
                            parts[zb]
                        )
                else:
                    recv = pltpu.make_async_remote_copy(
                        src_ref=buf.at[j, 3],
                        dst_ref=buf.at[j, s - 1],
                        send_sem=ssem.at[j, s - 1],
                        recv_sem=rsem.at[j, s - 1],
                        device_id=(src_nbr,),
                        device_id_type=pl.DeviceIdType.MESH,
                    )
                    recv.wait_recv()
                    for zb in range(Z):
                        buf[j, slot, zb * m_per : (zb + 1) * m_per, :] = (
                            buf[j, slot, zb * m_per : (zb + 1) * m_per, :]
                            + parts[zb]
                        )
                if s < Q - 1:
                    send = pltpu.make_async_remote_copy(
                        src_ref=buf.at[j, slot],
                        dst_ref=buf.at[j, s],
                        send_sem=ssem.at[j, s],
                        recv_sem=rsem.at[j, s],
                        device_id=(dst_nbr,),
                        device_id_type=pl.DeviceIdType.MESH,
                    )
                    send.start()
                    sends.append(send)

        def phase2_send(j):
            for k in range(Z - 1):
                dz = jnp.where(myz <= k, k + 1, k)
                dest = dz * Q + myq
                kr = jnp.where(myz > dz, myz - 1, myz)
                for h, buf in ((0, bufR), (1, bufL)):
                    send = pltpu.make_async_remote_copy(
                        src_ref=buf.at[j, 2, pl.ds(dz * m_per, m_per), :],
                        dst_ref=p2buf.at[
                            j, kr, :, pl.ds(h * half_w, half_w)
                        ],
                        send_sem=p2ssem.at[j, k, h],
                        recv_sem=p2rsem.at[j, kr, h],
                        device_id=(dest,),
                        device_id_type=pl.DeviceIdType.MESH,
                    )
                    send.start()
                    sends.append(send)

        def phase2_drain(j):
            col0 = j * slice_w
            for k in range(Z - 1):
                for h in range(2):
                    recv = pltpu.make_async_remote_copy(
                        src_ref=p2buf.at[j, k, :, pl.ds(h * half_w, half_w)],
                        dst_ref=p2buf.at[j, k, :, pl.ds(h * half_w, half_w)],
                        send_sem=p2ssem.at[j, k, h],
                        recv_sem=p2rsem.at[j, k, h],
                        device_id=(q_left,),
                        device_id_type=pl.DeviceIdType.MESH,
                    )
                    recv.wait_recv()
            for h, buf in ((0, bufR), (1, bufL)):
                acc = buf[j, 2, pl.ds(myz * m_per, m_per), :]
                for k in range(Z - 1):
                    acc = acc + p2buf[j, k, :, h * half_w : (h + 1) * half_w]
                out_ref[:, col0 + h * half_w : col0 + (h + 1) * half_w] = (
                    acc * jax.nn.sigmoid(acc)
                )

        for s in range(Q):
            for j in range(C):
                phase1_step(j, s)
                if s == Q - 1:
                    phase2_send(j)
        for j in range(C):
            phase2_drain(j)

        for send in sends:
            send.wait_send()

        @functools.partial(
            pl.run_scoped, second_barrier=pltpu.SemaphoreType.REGULAR
        )
        def _(second_barrier):
            for nbr in partners:
                pl.semaphore_signal(
                    second_barrier, inc=1,
                    device_id=(nbr,), device_id_type=pl.DeviceIdType.MESH,
                )
            pl.semaphore_wait(second_barrier, len(partners))

    return pl.pallas_call(
        body,
        out_shape=jax.ShapeDtypeStruct((m_per, n), jnp.float32),
        in_specs=[
            pl.BlockSpec(memory_space=pltpu.VMEM),
            pl.BlockSpec(memory_space=pltpu.VMEM),
        ],
        out_specs=pl.BlockSpec(memory_space=pltpu.VMEM),
        scratch_shapes=(
            [
                pltpu.VMEM((C, Q, Q * m_per, half_w), jnp.float32),
                pltpu.VMEM((C, Q, Q * m_per, half_w), jnp.float32),
                pltpu.SemaphoreType.DMA((C, Q - 1)),
                pltpu.SemaphoreType.DMA((C, Q - 1)),
                pltpu.SemaphoreType.DMA((C, Q - 1)),
                pltpu.SemaphoreType.DMA((C, Q - 1)),
                pltpu.VMEM((C, Z - 1, m_per, slice_w), jnp.float32),
                pltpu.SemaphoreType.DMA((C, Z - 1, 2)),
                pltpu.SemaphoreType.DMA((C, Z - 1, 2)),
            ]
        ),
        compiler_params=pltpu.CompilerParams(
            collective_id=0,
            vmem_limit_bytes=100 * 1024 * 1024,
        ),
    )(x, w_mat)


# device time: 169616 ns/iter; 1.2111x vs baseline; 1.2111x over previous
import functools

import jax
import jax.numpy as jnp
from jax import lax
from jax.experimental import pallas as pl
from jax.experimental.pallas import tpu as pltpu

N_DEV = 16
Z = 4
Q = 4
C = 8


def kernel(x, w_mat):
    m_glob, k_per = x.shape
    _, n = w_mat.shape
    m_per = m_glob // N_DEV
    slice_w = n // C
    half_w = slice_w // 2

    def body(x_ref, w_ref, out_ref, *scratch):
        bufR, bufL = scratch[0], scratch[1]
        ssemR, rsemR = scratch[2], scratch[3]
        ssemL, rsemL = scratch[4], scratch[5]
        p2buf = scratch[6]
        p2ssem, p2rsem = scratch[7], scratch[8]

        my = lax.axis_index("i")
        myz = my // Q
        myq = lax.rem(my, Q)
        plane_base = myz * Q
        q_right = plane_base + lax.rem(myq + 1, Q)
        q_left = plane_base + lax.rem(myq + Q - 1, Q)

        partners = [q_left, q_right]
        for k in range(Z - 1):
            dz = jnp.where(myz <= k, k + 1, k)
            partners.append(dz * Q + myq)
        barrier_sem = pltpu.get_barrier_semaphore()
        for nbr in partners:
            pl.semaphore_signal(
                barrier_sem, inc=1,
                device_id=(nbr,), device_id_type=pl.DeviceIdType.MESH,
            )
        pl.semaphore_wait(barrier_sem, len(partners))

        sends = []

        def phase1_step(j, s):
            col0 = j * slice_w
            for dirn in (+1, -1):
                if dirn == +1:
                    g = lax.rem(myq + 2 * Q - 1 - s, Q)
                    src_nbr, dst_nbr = q_left, q_right
                    buf, ssem, rsem = bufR, ssemR, rsemR
                    c0 = col0
                else:
                    g = lax.rem(myq + 1 + s, Q)
                    src_nbr, dst_nbr = q_right, q_left
                    buf, ssem, rsem = bufL, ssemL, rsemL
                    c0 = col0 + half_w
                parts = []
                for zb in range(Z):
                    b = zb * Q + g
                    parts.append(
                        jnp.dot(
                            x_ref[pl.ds(b * m_per, m_per), :],
                            w_ref[:, c0 : c0 + half_w],
                            preferred_element_type=jnp.float32,
                        )
                    )
                slot = 3 if s == 0 else s - 1
                if s == 0:
                    for zb in range(Z):
                        buf[j, slot, zb * m_per : (zb + 1) * m_per, :] = (
                            parts[zb]
                        )
                else:
                    recv = pltpu.make_async_remote_copy(
                        src_ref=buf.at[j, 3],
                        dst_ref=buf.at[j, s - 1],
                        send_sem=ssem.at[j, s - 1],
                        recv_sem=rsem.at[j, s - 1],
                        device_id=(src_nbr,),
                        device_id_type=pl.DeviceIdType.MESH,
                    )
                    recv.wait_recv()
                    for zb in range(Z):
                        buf[j, slot, zb * m_per : (zb + 1) * m_per, :] = (
                            buf[j, slot, zb * m_per : (zb + 1) * m_per, :]
                            + parts[zb]
                        )
                if s < Q - 1:
                    send = pltpu.make_async_remote_copy(
                        src_ref=buf.at[j, slot],
                        dst_ref=buf.at[j, s],
                        send_sem=ssem.at[j, s],
                        recv_sem=rsem.at[j, s],
                        device_id=(dst_nbr,),
                        device_id_type=pl.DeviceIdType.MESH,
                    )
                    send.start()
                    sends.append(send)

        def phase2_send(j):
            for k in range(Z - 1):
                dz = jnp.where(myz <= k, k + 1, k)
                dest = dz * Q + myq
                kr = jnp.where(myz > dz, myz - 1, myz)
                for h, buf in ((0, bufR), (1, bufL)):
                    send = pltpu.make_async_remote_copy(
                        src_ref=buf.at[j, 2, pl.ds(dz * m_per, m_per), :],
                        dst_ref=p2buf.at[
                            j, kr, :, pl.ds(h * half_w, half_w)
                        ],
                        send_sem=p2ssem.at[j, k, h],
                        recv_sem=p2rsem.at[j, kr, h],
                        device_id=(dest,),
                        device_id_type=pl.DeviceIdType.MESH,
                    )
                    send.start()
                    sends.append(send)

        def phase2_drain(j):
            col0 = j * slice_w
            for k in range(Z - 1):
                for h in range(2):
                    recv = pltpu.make_async_remote_copy(
                        src_ref=p2buf.at[j, k, :, pl.ds(h * half_w, half_w)],
                        dst_ref=p2buf.at[j, k, :, pl.ds(h * half_w, half_w)],
                        send_sem=p2ssem.at[j, k, h],
                        recv_sem=p2rsem.at[j, k, h],
                        device_id=(q_left,),
                        device_id_type=pl.DeviceIdType.MESH,
                    )
                    recv.wait_recv()
            for h, buf in ((0, bufR), (1, bufL)):
                acc = buf[j, 2, pl.ds(myz * m_per, m_per), :]
                for k in range(Z - 1):
                    acc = acc + p2buf[j, k, :, h * half_w : (h + 1) * half_w]
                out_ref[:, col0 + h * half_w : col0 + (h + 1) * half_w] = (
                    acc * jax.nn.sigmoid(acc)
                )

        DRAIN_LAG = 2
        for t in range(C + Q - 1 + DRAIN_LAG):
            for j in range(max(0, t - (Q - 1)), min(C, t + 1)):
                s = t - j
                phase1_step(j, s)
                if s == Q - 1:
                    phase2_send(j)
            jd = t - (Q - 1) - DRAIN_LAG
            if 0 <= jd < C:
                phase2_drain(jd)

        for send in sends:
            send.wait_send()

        @functools.partial(
            pl.run_scoped, second_barrier=pltpu.SemaphoreType.REGULAR
        )
        def _(second_barrier):
            for nbr in partners:
                pl.semaphore_signal(
                    second_barrier, inc=1,
                    device_id=(nbr,), device_id_type=pl.DeviceIdType.MESH,
                )
            pl.semaphore_wait(second_barrier, len(partners))

    return pl.pallas_call(
        body,
        out_shape=jax.ShapeDtypeStruct((m_per, n), jnp.float32),
        in_specs=[
            pl.BlockSpec(memory_space=pltpu.VMEM),
            pl.BlockSpec(memory_space=pltpu.VMEM),
        ],
        out_specs=pl.BlockSpec(memory_space=pltpu.VMEM),
        scratch_shapes=(
            [
                pltpu.VMEM((C, Q, Q * m_per, half_w), jnp.float32),
                pltpu.VMEM((C, Q, Q * m_per, half_w), jnp.float32),
                pltpu.SemaphoreType.DMA((C, Q - 1)),
                pltpu.SemaphoreType.DMA((C, Q - 1)),
                pltpu.SemaphoreType.DMA((C, Q - 1)),
                pltpu.SemaphoreType.DMA((C, Q - 1)),
                pltpu.VMEM((C, Z - 1, m_per, slice_w), jnp.float32),
                pltpu.SemaphoreType.DMA((C, Z - 1, 2)),
                pltpu.SemaphoreType.DMA((C, Z - 1, 2)),
            ]
        ),
        compiler_params=pltpu.CompilerParams(
            collective_id=0,
            vmem_limit_bytes=100 * 1024 * 1024,
        ),
    )(x, w_mat)
